# baseline (speedup 1.0000x reference)
import functools

import jax
import jax.numpy as jnp
from jax import lax
from jax.experimental import pallas as pl
from jax.experimental.pallas import tpu as pltpu
from jax.experimental.pallas import tpu_sc as plsc

B, C, H, W = 8, 192, 224, 224
HW = H * W
BC = B * C
K = 19
ROW_BLK = 48
ZERO_BLK = 512


def _sum_copy_kernel(x_ref, o_ref, c_ref):
    o_ref[:, 0] = jnp.sum(x_ref[...], axis=(1, 2))
    c_ref[...] = x_ref[...]


def _prob_kernel(s_ref, p_ref):
    mean = s_ref[...] / float(HW)
    g = mean ** 2
    s = jnp.sqrt(jnp.abs(g))
    mx = jnp.max(s, axis=1, keepdims=True)
    p_ref[...] = s / mx


NCHUNK = C // 16


def _sc_mask_kernel(p_hbm, u_hbm, keep_hbm, prow, urow, work, low, newp, kout):
    wid = lax.axis_index("s") * 2 + lax.axis_index("c")

    @pl.when(wid < B)
    def _():
        pltpu.sync_copy(p_hbm.at[wid], prow)
        pltpu.sync_copy(u_hbm.at[wid], urow)
        for j in range(NCHUNK):
            c = prow[pl.ds(j * 16, 16)]
            work[pl.ds(j * 16, 16)] = c
            low[pl.ds(j * 16, 16)] = c
            newp[pl.ds(j * 16, 16)] = c

        def body(_, carry):
            mvec = jnp.full((16,), -jnp.inf, jnp.float32)
            nvec = jnp.full((16,), jnp.inf, jnp.float32)
            for j in range(NCHUNK):
                mvec = jnp.maximum(mvec, work[pl.ds(j * 16, 16)])
                nvec = jnp.minimum(nvec, low[pl.ds(j * 16, 16)])
            top_v = jnp.max(mvec)
            btm_v = jnp.min(nvec)
            avec = jnp.full((16,), C, jnp.int32)
            ivec = jnp.full((16,), C, jnp.int32)
            for j in range(NCHUNK):
                idx = lax.iota(jnp.int32, 16) + (j * 16)
                avec = jnp.minimum(
                    avec, jnp.where(work[pl.ds(j * 16, 16)] == top_v, idx, C))
                ivec = jnp.minimum(
                    ivec, jnp.where(low[pl.ds(j * 16, 16)] == btm_v, idx, C))
            amax = jnp.min(avec)
            amin = jnp.min(ivec)
            scal = top_v - (top_v - btm_v)
            for j in range(NCHUNK):
                idx = lax.iota(jnp.int32, 16) + (j * 16)
                sel = idx == amax
                newp[pl.ds(j * 16, 16)] = jnp.where(
                    sel, scal, newp[pl.ds(j * 16, 16)])
                work[pl.ds(j * 16, 16)] = jnp.where(
                    sel, -jnp.inf, work[pl.ds(j * 16, 16)])
                low[pl.ds(j * 16, 16)] = jnp.where(
                    idx == amin, jnp.inf, low[pl.ds(j * 16, 16)])
            return carry

        jax.lax.fori_loop(0, K, body, 0)

        for j in range(NCHUNK):
            np_c = jnp.clip(newp[pl.ds(j * 16, 16)], 0.0, 1.0)
            kout[pl.ds(j * 16, 16)] = (
                urow[pl.ds(j * 16, 16)] < (1.0 - np_c)).astype(jnp.int32)
        pltpu.sync_copy(kout, keep_hbm.at[wid])


def _zero_kernel(keep_ref, x_any, o_any, zbuf, sems):
    del x_any
    i = pl.program_id(0)

    @pl.when(i == 0)
    def _():
        zbuf[...] = jnp.zeros_like(zbuf)

    for r in range(ZERO_BLK):
        row = i * ZERO_BLK + r

        @pl.when(keep_ref[row] == 0)
        def _():
            pltpu.make_async_copy(zbuf, o_any.at[row], sems.at[r]).start()

    for r in range(ZERO_BLK):
        row = i * ZERO_BLK + r

        @pl.when(keep_ref[row] == 0)
        def _():
            pltpu.make_async_copy(zbuf, o_any.at[row], sems.at[r]).wait()


def kernel(x):
    x3 = x.reshape(BC, H, W)

    sums, xcopy = pl.pallas_call(
        _sum_copy_kernel,
        grid=(BC // ROW_BLK,),
        in_specs=[pl.BlockSpec((ROW_BLK, H, W), lambda i: (i, 0, 0))],
        out_specs=[
            pl.BlockSpec((ROW_BLK, 1), lambda i: (i, 0)),
            pl.BlockSpec((ROW_BLK, H, W), lambda i: (i, 0, 0)),
        ],
        out_shape=[
            jax.ShapeDtypeStruct((BC, 1), jnp.float32),
            jax.ShapeDtypeStruct((BC, H, W), jnp.float32),
        ],
        compiler_params=pltpu.CompilerParams(
            dimension_semantics=("parallel",)),
    )(x3)

    u = jax.random.uniform(jax.random.key(42), (B, C), jnp.float32)

    p = pl.pallas_call(
        _prob_kernel,
        out_shape=jax.ShapeDtypeStruct((B, C), jnp.float32),
    )(sums.reshape(B, C))

    sc_mask = functools.partial(
        pl.kernel,
        out_type=jax.ShapeDtypeStruct((B, C), jnp.int32),
        mesh=plsc.VectorSubcoreMesh(core_axis_name="c", subcore_axis_name="s"),
        compiler_params=pltpu.CompilerParams(needs_layout_passes=False),
        scratch_types=[
            pltpu.MemorySpace.VMEM((C,), jnp.float32),
            pltpu.MemorySpace.VMEM((C,), jnp.float32),
            pltpu.MemorySpace.VMEM((C,), jnp.float32),
            pltpu.MemorySpace.VMEM((C,), jnp.float32),
            pltpu.MemorySpace.VMEM((C,), jnp.float32),
            pltpu.MemorySpace.VMEM((C,), jnp.int32),
        ],
    )
    keep = sc_mask(_sc_mask_kernel)(p, u)

    out = pl.pallas_call(
        _zero_kernel,
        grid_spec=pltpu.PrefetchScalarGridSpec(
            num_scalar_prefetch=1,
            grid=(BC // ZERO_BLK,),
            in_specs=[pl.BlockSpec(memory_space=pl.ANY)],
            out_specs=pl.BlockSpec(memory_space=pl.ANY),
            scratch_shapes=[
                pltpu.MemorySpace.VMEM((H, W), jnp.float32),
                pltpu.SemaphoreType.DMA((ZERO_BLK,)),
            ],
        ),
        out_shape=jax.ShapeDtypeStruct((BC, H, W), jnp.float32),
        input_output_aliases={1: 0},
    )(keep.reshape(BC), xcopy)

    return out.reshape(B, C, H, W)
